# Initial kernel scaffold; baseline (speedup 1.0000x reference)
import jax
import jax.numpy as jnp
from jax.experimental import pallas as pl


def kernel(interaction_pairs, emb_table, lin_table, bias):
    raise NotImplementedError("write your pallas kernel here")


# speedup vs baseline: 25.5197x; 25.5197x over previous
import jax
import jax.numpy as jnp
from jax import lax
from jax.experimental import pallas as pl
from jax.experimental.pallas import tpu as pltpu
from jax.experimental.pallas import tpu_sc as plsc

BATCH = 16384
FIELDS = 100
EMBED_DIM = 128
VOCAB = 100000

NC = 2
NS = 16
NW = NC * NS
BPW = BATCH // NW
CH = 64
NCHUNK = BPW // CH
NV = EMBED_DIM // 16
LINPAD = VOCAB + 16


def _fm_body(idx_hbm, emb_hbm, lin_hbm, out_hbm,
             idx_v, rows_v, lin_v, out_v, sem_e, sem_i):
    wid = lax.axis_index("s") * NC + lax.axis_index("c")
    base = wid * BPW

    pltpu.sync_copy(lin_hbm, lin_v)
    bvec = lin_v[pl.ds(VOCAB, 16)]
    lanes = lax.iota(jnp.int32, 16)
    lane0 = lanes == 0
    zeros = jnp.zeros((16,), jnp.float32)

    def item_body(j, ci):
        pltpu.async_copy(emb_hbm.at[idx_v.at[j]], rows_v, sem_e).wait()

        def row_acc(r, carry):
            new = []
            for v in range(NV):
                x = rows_v[r, pl.ds(v * 16, 16)]
                new.append(carry[v] + x)
                new.append(carry[NV + v] + x * x)
            return tuple(new[0::2]) + tuple(new[1::2])

        accs = lax.fori_loop(0, FIELDS, row_acc, (zeros,) * (2 * NV))

        lsum = zeros
        for v in range(FIELDS // 16):
            g = plsc.load_gather(lin_v, [idx_v[j, pl.ds(v * 16, 16)]])
            lsum = lsum + g
        g = plsc.load_gather(lin_v, [idx_v[j, pl.ds(FIELDS - 16, 16)]])
        lsum = lsum + jnp.where(lanes >= 12, g, zeros)

        t = zeros
        for v in range(NV):
            s = accs[v]
            t = t + (s * s - accs[NV + v])
        rvec = 0.5 * t + lsum + bvec
        res = jnp.full((16,), jnp.sum(rvec), jnp.float32)
        posv = jnp.full((16,), ci * CH, jnp.int32) + j
        plsc.store_scatter(out_v, [posv], res, mask=lane0)
        return ci

    for ci in range(NCHUNK):
        pltpu.sync_copy(idx_hbm.at[pl.ds(base + ci * CH, CH)], idx_v)
        lax.fori_loop(0, CH, item_body, ci)

    pltpu.sync_copy(out_v, out_hbm.at[pl.ds(base, BPW)])


def kernel(interaction_pairs, emb_table, lin_table, bias):
    lin_aug = jnp.concatenate(
        [lin_table.reshape((VOCAB,)), bias,
         jnp.zeros((LINPAD - VOCAB - 1,), jnp.float32)])
    mesh = plsc.VectorSubcoreMesh(core_axis_name="c", subcore_axis_name="s")
    fm = pl.kernel(
        _fm_body,
        out_type=jax.ShapeDtypeStruct((BATCH,), jnp.float32),
        mesh=mesh,
        scratch_types=[
            pltpu.VMEM((CH, FIELDS), jnp.int32),
            pltpu.VMEM((FIELDS, EMBED_DIM), jnp.float32),
            pltpu.VMEM((LINPAD,), jnp.float32),
            pltpu.VMEM((BPW,), jnp.float32),
            pltpu.SemaphoreType.DMA,
            pltpu.SemaphoreType.DMA,
        ],
        compiler_params=pltpu.CompilerParams(needs_layout_passes=False),
    )
    return fm(interaction_pairs, emb_table, lin_aug)


# speedup vs baseline: 42.1626x; 1.6522x over previous
import jax
import jax.numpy as jnp
from jax import lax
from jax.experimental import pallas as pl
from jax.experimental.pallas import tpu as pltpu
from jax.experimental.pallas import tpu_sc as plsc

BATCH = 16384
FIELDS = 100
EMBED_DIM = 128
VOCAB = 100000

NC = 2
NS = 16
NW = NC * NS
BPW = BATCH // NW
CH = 16
NCHUNK = BPW // CH
NV = EMBED_DIM // 16
LINPAD = VOCAB + 16


def _fm_body(idx_hbm, emb_hbm, lin_hbm, out_hbm,
             idx_v, rows0_v, rows1_v, lin_v, out_v, sem0, sem1, sem_i):
    wid = lax.axis_index("s") * NC + lax.axis_index("c")
    base = wid * BPW

    pltpu.sync_copy(lin_hbm, lin_v)
    bvec = lin_v[pl.ds(VOCAB, 16)]
    lanes = lax.iota(jnp.int32, 16)
    lane0 = lanes == 0
    zeros = jnp.zeros((16,), jnp.float32)
    sems = (sem0, sem1)
    rows = (rows0_v, rows1_v)

    def fire(j, b):
        pltpu.async_copy(emb_hbm.at[idx_v.at[j]], rows[b], sems[b])

    def wait(b):
        pltpu.make_async_copy(emb_hbm.at[idx_v.at[0]], rows[b],
                              sems[b]).wait()

    def compute(j, b, ci):
        def row_acc(r, carry):
            new = []
            for v in range(NV):
                x = rows[b][r, pl.ds(v * 16, 16)]
                new.append(carry[v] + x)
                new.append(carry[NV + v] + x * x)
            return tuple(new[0::2]) + tuple(new[1::2])

        accs = lax.fori_loop(0, FIELDS, row_acc, (zeros,) * (2 * NV))

        lsum = zeros
        for v in range(FIELDS // 16):
            g = plsc.load_gather(lin_v, [idx_v[j, pl.ds(v * 16, 16)]])
            lsum = lsum + g
        g = plsc.load_gather(lin_v, [idx_v[j, pl.ds(FIELDS - 16, 16)]])
        lsum = lsum + jnp.where(lanes >= 12, g, zeros)

        t = zeros
        for v in range(NV):
            s = accs[v]
            t = t + (s * s - accs[NV + v])
        rvec = 0.5 * t + lsum + bvec
        res = jnp.full((16,), jnp.sum(rvec), jnp.float32)
        posv = jnp.full((16,), ci * CH, jnp.int32) + j
        plsc.store_scatter(out_v, [posv], res, mask=lane0)

    for ci in range(NCHUNK):
        pltpu.sync_copy(idx_hbm.at[pl.ds(base + ci * CH, CH)], idx_v)
        fire(0, 0)

        def pair_body(p, _):
            j0 = 2 * p
            j1 = j0 + 1
            fire(j1, 1)
            wait(0)
            compute(j0, 0, ci)

            @pl.when(p < CH // 2 - 1)
            def _():
                fire(j0 + 2, 0)

            wait(1)
            compute(j1, 1, ci)
            return 0

        lax.fori_loop(0, CH // 2, pair_body, 0)

    pltpu.sync_copy(out_v, out_hbm.at[pl.ds(base, BPW)])


def kernel(interaction_pairs, emb_table, lin_table, bias):
    lin_aug = jnp.concatenate(
        [lin_table.reshape((VOCAB,)), bias,
         jnp.zeros((LINPAD - VOCAB - 1,), jnp.float32)])
    mesh = plsc.VectorSubcoreMesh(core_axis_name="c", subcore_axis_name="s")
    fm = pl.kernel(
        _fm_body,
        out_type=jax.ShapeDtypeStruct((BATCH,), jnp.float32),
        mesh=mesh,
        scratch_types=[
            pltpu.VMEM((CH, FIELDS), jnp.int32),
            pltpu.VMEM((FIELDS, EMBED_DIM), jnp.float32),
            pltpu.VMEM((FIELDS, EMBED_DIM), jnp.float32),
            pltpu.VMEM((LINPAD,), jnp.float32),
            pltpu.VMEM((BPW,), jnp.float32),
            pltpu.SemaphoreType.DMA,
            pltpu.SemaphoreType.DMA,
            pltpu.SemaphoreType.DMA,
        ],
        compiler_params=pltpu.CompilerParams(needs_layout_passes=False),
    )
    return fm(interaction_pairs, emb_table, lin_aug)
